# Initial kernel scaffold; baseline (speedup 1.0000x reference)
import functools

import jax
import jax.numpy as jnp
from jax import lax
from jax.experimental import pallas as pl
from jax.experimental.pallas import tpu as pltpu
from jax.experimental.pallas import tpu_sc as plsc

_N = 10000
_H = 4
_CH = 128
_NC = 2
_NS = 16
_NW = _NC * _NS
_HIGH = lax.Precision.HIGHEST


def _cdiv(a, b):
    return (a + b - 1) // b


def _sc_mesh():
    return plsc.VectorSubcoreMesh(core_axis_name="c", subcore_axis_name="s")


@functools.partial(jax.jit, static_argnames=("epad",))
def _sc_degree(dstp, zeros16, ones16, *, epad):
    chunks = epad // (_NW * _CH)
    rows = _N // _NS

    @functools.partial(
        pl.kernel,
        out_type=jax.ShapeDtypeStruct((_NC, _N, 16), jnp.float32),
        mesh=_sc_mesh(),
        scratch_types=[
            pltpu.VMEM_SHARED((_N + 1, 16), jnp.float32),
            pltpu.VMEM((_CH, 16), jnp.float32),
            pltpu.VMEM((_CH, 16), jnp.float32),
            pltpu.VMEM((_CH,), jnp.int32),
        ],
    )
    def degk(dst_hbm, z_hbm, o_hbm, out_hbm, acc_sh, zb, ob, idxb):
        cid = lax.axis_index("c")
        sid = lax.axis_index("s")
        pltpu.sync_copy(z_hbm, zb)
        pltpu.sync_copy(o_hbm, ob)
        for j in range(5):
            pltpu.sync_copy(
                zb.at[pl.ds(0, 125)],
                acc_sh.at[pl.ds(sid * rows + j * 125, 125)],
            )
        plsc.subcore_barrier()
        ebase = (cid * _NS + sid) * (chunks * _CH)

        def body(j, carry):
            pltpu.sync_copy(dst_hbm.at[pl.ds(ebase + j * _CH, _CH)], idxb)
            pltpu.sync_copy(ob, acc_sh.at[idxb], add=True)
            return carry

        lax.fori_loop(0, chunks, body, 0)
        plsc.subcore_barrier()
        pltpu.sync_copy(
            acc_sh.at[pl.ds(sid * rows, rows)],
            out_hbm.at[cid, pl.ds(sid * rows, rows)],
        )

    return degk(dstp, zeros16, ones16)


@functools.partial(jax.jit, static_argnames=("epad", "d"))
def _sc_aggregate(hs, srcp, dstp, zrows, *, epad, d):
    chunks = epad // (_NW * _CH)
    rows = _N // _NS

    @functools.partial(
        pl.kernel,
        out_type=jax.ShapeDtypeStruct((_NC, _N, d), jnp.float32),
        mesh=_sc_mesh(),
        scratch_types=[
            pltpu.VMEM_SHARED((_N + 1, d), jnp.float32),
            pltpu.VMEM((_CH, d), jnp.float32),
            pltpu.VMEM((_CH, d), jnp.float32),
            pltpu.VMEM((_CH,), jnp.int32),
            pltpu.VMEM((_CH,), jnp.int32),
            pltpu.SemaphoreType.DMA,
        ],
    )
    def aggk(hs_hbm, src_hbm, dst_hbm, z_hbm, out_hbm,
             acc_sh, zb, rowsb, sidxb, didxb, sem):
        cid = lax.axis_index("c")
        sid = lax.axis_index("s")
        pltpu.sync_copy(z_hbm, zb)
        for j in range(5):
            pltpu.sync_copy(
                zb.at[pl.ds(0, 125)],
                acc_sh.at[pl.ds(sid * rows + j * 125, 125)],
            )
        plsc.subcore_barrier()
        ebase = (cid * _NS + sid) * (chunks * _CH)

        def body(j, carry):
            base = ebase + j * _CH
            pltpu.sync_copy(src_hbm.at[pl.ds(base, _CH)], sidxb)
            pltpu.sync_copy(dst_hbm.at[pl.ds(base, _CH)], didxb)
            pltpu.async_copy(hs_hbm.at[sidxb], rowsb, sem).wait()
            pltpu.sync_copy(rowsb, acc_sh.at[didxb], add=True)
            return carry

        lax.fori_loop(0, chunks, body, 0)
        plsc.subcore_barrier()
        pltpu.sync_copy(
            acc_sh.at[pl.ds(sid * rows, rows)],
            out_hbm.at[cid, pl.ds(sid * rows, rows)],
        )

    return aggk(hs, srcp, dstp, zrows)


def _tc1_body(x_ref, w0_ref, degp_ref, dinv_ref, hs0_ref):
    deg = degp_ref[0][:, 0:1] + degp_ref[1][:, 0:1] + 1.0
    dinv = lax.rsqrt(deg)
    dinv_ref[...] = dinv
    h1 = jnp.dot(x_ref[...], w0_ref[...], precision=_HIGH)
    hs0_ref[0:_N, :] = h1 * dinv


def _tc1(x, w0, degp):
    return pl.pallas_call(
        _tc1_body,
        out_shape=[
            jax.ShapeDtypeStruct((_N, 1), jnp.float32),
            jax.ShapeDtypeStruct((_N + 1, w0.shape[1]), jnp.float32),
        ],
    )(x, w0, degp)


def _layer_tail(aggp_ref, hs_ref, dinv_ref, res_in_ref, gb_ref, wbd_ref,
                cb_ref, g_ref, b_ref, wrt_ref, rb_ref):
    dinv = dinv_ref[...]
    agg = aggp_ref[0] + aggp_ref[1] + hs_ref[0:_N, :]
    h = dinv * agg + gb_ref[...]
    h = jnp.maximum(h, 0.0)
    hc = jnp.dot(h, wbd_ref[...], precision=_HIGH) + cb_ref[...]
    m = jnp.mean(hc, axis=0, keepdims=True)
    dlt = hc - m
    v = jnp.mean(dlt * dlt, axis=0, keepdims=True)
    hb = g_ref[...] * (dlt / jnp.sqrt(v + 1e-5)) + b_ref[...]
    res = jnp.dot(res_in_ref[...], wrt_ref[...], precision=_HIGH) + rb_ref[...]
    return hb + res


def _tc2_body(aggp_ref, hs0_ref, dinv_ref, x_ref, gb_ref, wbd_ref, cb_ref,
              g_ref, b_ref, wrt_ref, rb_ref, w1_ref, h0_ref, hs1_ref):
    h0 = _layer_tail(aggp_ref, hs0_ref, dinv_ref, x_ref, gb_ref, wbd_ref,
                     cb_ref, g_ref, b_ref, wrt_ref, rb_ref)
    h0_ref[...] = h0
    h1 = jnp.dot(h0, w1_ref[...], precision=_HIGH)
    hs1_ref[0:_N, :] = h1 * dinv_ref[...]


def _tc2(aggp0, hs0, dinv, x, gb0, wbd0, cb0, g0, b0, wr0t, rb0, w1):
    return pl.pallas_call(
        _tc2_body,
        out_shape=[
            jax.ShapeDtypeStruct((_N, wbd0.shape[1]), jnp.float32),
            jax.ShapeDtypeStruct((_N + 1, w1.shape[1]), jnp.float32),
        ],
    )(aggp0, hs0, dinv, x, gb0, wbd0, cb0, g0, b0, wr0t, rb0, w1)


def _tc3_body(aggp_ref, hs1_ref, dinv_ref, h0_ref, gb_ref, wbd_ref, cb_ref,
              g_ref, b_ref, wrt_ref, rb_ref, out_ref):
    out_ref[...] = _layer_tail(aggp_ref, hs1_ref, dinv_ref, h0_ref, gb_ref,
                               wbd_ref, cb_ref, g_ref, b_ref, wrt_ref, rb_ref)


def _tc3(aggp1, hs1, dinv, h0, gb1, wbd1, cb1, g1, b1, wr1t, rb1):
    return pl.pallas_call(
        _tc3_body,
        out_shape=jax.ShapeDtypeStruct((_N, wbd1.shape[1]), jnp.float32),
    )(aggp1, hs1, dinv, h0, gb1, wbd1, cb1, g1, b1, wr1t, rb1)


def _blockdiag(wc):
    dout = wc.shape[0]
    g = dout // _H
    wg = wc.reshape(g, _H, _H)
    eye = jnp.eye(g, dtype=wc.dtype)
    bd = eye[:, None, :, None] * wg.transpose(0, 2, 1)[:, :, None, :]
    return bd.reshape(dout, dout)


def kernel(x, edge_index, W0, gb0, Wc0, cb0, g0, b0, Wr0, rb0,
           W1, gb1, Wc1, cb1, g1, b1, Wr1, rb1):
    e = edge_index.shape[1]
    per = _NW * _CH
    epad = _cdiv(e, per) * per
    pad = epad - e
    fill = jnp.full((pad,), _N, dtype=jnp.int32)
    srcp = jnp.concatenate([edge_index[0], fill])
    dstp = jnp.concatenate([edge_index[1], fill])

    zeros16 = jnp.zeros((_CH, 16), jnp.float32)
    ones16 = jnp.ones((_CH, 16), jnp.float32)
    zeros64 = jnp.zeros((_CH, 64), jnp.float32)
    zeros32 = jnp.zeros((_CH, 32), jnp.float32)

    row = lambda v: v.reshape(1, -1)
    wbd0 = _blockdiag(Wc0)
    wbd1 = _blockdiag(Wc1)

    degp = _sc_degree(dstp, zeros16, ones16, epad=epad)
    dinv, hs0 = _tc1(x, W0, degp)
    aggp0 = _sc_aggregate(hs0, srcp, dstp, zeros64, epad=epad, d=64)
    h0, hs1 = _tc2(aggp0, hs0, dinv, x, row(gb0), wbd0, row(cb0), row(g0),
                   row(b0), Wr0.T, row(rb0), W1)
    aggp1 = _sc_aggregate(hs1, srcp, dstp, zeros32, epad=epad, d=32)
    return _tc3(aggp1, hs1, dinv, h0, row(gb1), wbd1, row(cb1), row(g1),
                row(b1), Wr1.T, row(rb1))


# speedup vs baseline: 15.4558x; 15.4558x over previous
import functools

import jax
import jax.numpy as jnp
from jax import lax
from jax.experimental import pallas as pl
from jax.experimental.pallas import tpu as pltpu
from jax.experimental.pallas import tpu_sc as plsc

_N = 10000
_H = 4
_CH = 128
_NC = 2
_NS = 16
_NW = _NC * _NS
_NPAD = 10112
_RPT = _NPAD // _NS
_HIGH = lax.Precision.HIGHEST
_TC_PARAMS = pltpu.CompilerParams(vmem_limit_bytes=100 * 1024 * 1024)


def _cdiv(a, b):
    return (a + b - 1) // b


def _sc_mesh():
    return plsc.VectorSubcoreMesh(core_axis_name="c", subcore_axis_name="s")


@functools.partial(jax.jit, static_argnames=("epad",))
def _sc_degree(dstp, zeros16, ones16, *, epad):
    chunks = epad // (_NW * _CH)

    @functools.partial(
        pl.kernel,
        out_type=jax.ShapeDtypeStruct((_NC, _NPAD, 16), jnp.float32),
        mesh=_sc_mesh(),
        compiler_params=pltpu.CompilerParams(use_tc_tiling_on_sc=False),
        scratch_types=[
            pltpu.VMEM_SHARED((_NPAD, 16), jnp.float32),
            pltpu.VMEM((_CH, 16), jnp.float32),
            pltpu.VMEM((_CH, 16), jnp.float32),
            pltpu.VMEM((_CH,), jnp.int32),
        ],
    )
    def degk(dst_hbm, z_hbm, o_hbm, out_hbm, acc_sh, zb, ob, idxb):
        cid = lax.axis_index("c")
        sid = lax.axis_index("s")
        pltpu.sync_copy(z_hbm, zb)
        pltpu.sync_copy(o_hbm, ob)
        for j, w in enumerate((128, 128, 128, 128, 120)):
            pltpu.sync_copy(
                zb.at[pl.ds(0, w)],
                acc_sh.at[pl.ds(sid * _RPT + j * 128, w)],
            )
        plsc.subcore_barrier()
        ebase = (cid * _NS + sid) * (chunks * _CH)

        def body(j, carry):
            pltpu.sync_copy(dst_hbm.at[pl.ds(ebase + j * _CH, _CH)], idxb)
            pltpu.sync_copy(ob, acc_sh.at[idxb], add=True)
            return carry

        lax.fori_loop(0, chunks, body, 0)
        plsc.subcore_barrier()
        pltpu.sync_copy(
            acc_sh.at[pl.ds(sid * _RPT, _RPT)],
            out_hbm.at[cid, pl.ds(sid * _RPT, _RPT)],
        )

    return degk(dstp, zeros16, ones16)


@functools.partial(jax.jit, static_argnames=("epad", "d"))
def _sc_aggregate(hs, srcp, dstp, zrows, *, epad, d):
    chunks = epad // (_NW * _CH)

    @functools.partial(
        pl.kernel,
        out_type=jax.ShapeDtypeStruct((_NC, _NPAD, d), jnp.float32),
        mesh=_sc_mesh(),
        compiler_params=pltpu.CompilerParams(use_tc_tiling_on_sc=False),
        scratch_types=[
            pltpu.VMEM_SHARED((_NPAD, d), jnp.float32),
            pltpu.VMEM((_CH, d), jnp.float32),
            pltpu.VMEM((_CH, d), jnp.float32),
            pltpu.VMEM((_CH,), jnp.int32),
            pltpu.VMEM((_CH,), jnp.int32),
            pltpu.SemaphoreType.DMA,
        ],
    )
    def aggk(hs_hbm, src_hbm, dst_hbm, z_hbm, out_hbm,
             acc_sh, zb, rowsb, sidxb, didxb, sem):
        cid = lax.axis_index("c")
        sid = lax.axis_index("s")
        pltpu.sync_copy(z_hbm, zb)
        for j, w in enumerate((128, 128, 128, 128, 120)):
            pltpu.sync_copy(
                zb.at[pl.ds(0, w)],
                acc_sh.at[pl.ds(sid * _RPT + j * 128, w)],
            )
        plsc.subcore_barrier()
        ebase = (cid * _NS + sid) * (chunks * _CH)

        def body(j, carry):
            base = ebase + j * _CH
            pltpu.sync_copy(src_hbm.at[pl.ds(base, _CH)], sidxb)
            pltpu.sync_copy(dst_hbm.at[pl.ds(base, _CH)], didxb)
            pltpu.async_copy(hs_hbm.at[sidxb], rowsb, sem).wait()
            pltpu.sync_copy(rowsb, acc_sh.at[didxb], add=True)
            return carry

        lax.fori_loop(0, chunks, body, 0)
        plsc.subcore_barrier()
        pltpu.sync_copy(
            acc_sh.at[pl.ds(sid * _RPT, _RPT)],
            out_hbm.at[cid, pl.ds(sid * _RPT, _RPT)],
        )

    return aggk(hs, srcp, dstp, zrows)


def _tc1_body(x_ref, w0_ref, degp_ref, dinv_ref, hs0_ref):
    deg = degp_ref[0][0:_N, 0:1] + degp_ref[1][0:_N, 0:1] + 1.0
    dinv = lax.rsqrt(deg)
    dinv_ref[...] = dinv
    h1 = jnp.dot(x_ref[...], w0_ref[...], precision=_HIGH)
    hs0_ref[0:_N, :] = h1 * dinv


def _tc1(x, w0, degp):
    return pl.pallas_call(
        _tc1_body,
        compiler_params=_TC_PARAMS,
        out_shape=[
            jax.ShapeDtypeStruct((_N, 1), jnp.float32),
            jax.ShapeDtypeStruct((_N + 1, w0.shape[1]), jnp.float32),
        ],
    )(x, w0, degp)


def _layer_tail(aggp_ref, hs_ref, dinv_ref, res_in_ref, gb_ref, wbd_ref,
                cb_ref, g_ref, b_ref, wrt_ref, rb_ref):
    dinv = dinv_ref[...]
    agg = aggp_ref[0][0:_N, :] + aggp_ref[1][0:_N, :] + hs_ref[0:_N, :]
    h = dinv * agg + gb_ref[...]
    h = jnp.maximum(h, 0.0)
    hc = jnp.dot(h, wbd_ref[...], precision=_HIGH) + cb_ref[...]
    m = jnp.mean(hc, axis=0, keepdims=True)
    dlt = hc - m
    v = jnp.mean(dlt * dlt, axis=0, keepdims=True)
    hb = g_ref[...] * (dlt / jnp.sqrt(v + 1e-5)) + b_ref[...]
    res = jnp.dot(res_in_ref[...], wrt_ref[...], precision=_HIGH) + rb_ref[...]
    return hb + res


def _tc2_body(aggp_ref, hs0_ref, dinv_ref, x_ref, gb_ref, wbd_ref, cb_ref,
              g_ref, b_ref, wrt_ref, rb_ref, w1_ref, h0_ref, hs1_ref):
    h0 = _layer_tail(aggp_ref, hs0_ref, dinv_ref, x_ref, gb_ref, wbd_ref,
                     cb_ref, g_ref, b_ref, wrt_ref, rb_ref)
    h0_ref[...] = h0
    h1 = jnp.dot(h0, w1_ref[...], precision=_HIGH)
    hs1_ref[0:_N, :] = h1 * dinv_ref[...]


def _tc2(aggp0, hs0, dinv, x, gb0, wbd0, cb0, g0, b0, wr0t, rb0, w1):
    return pl.pallas_call(
        _tc2_body,
        compiler_params=_TC_PARAMS,
        out_shape=[
            jax.ShapeDtypeStruct((_N, wbd0.shape[1]), jnp.float32),
            jax.ShapeDtypeStruct((_N + 1, w1.shape[1]), jnp.float32),
        ],
    )(aggp0, hs0, dinv, x, gb0, wbd0, cb0, g0, b0, wr0t, rb0, w1)


def _tc3_body(aggp_ref, hs1_ref, dinv_ref, h0_ref, gb_ref, wbd_ref, cb_ref,
              g_ref, b_ref, wrt_ref, rb_ref, out_ref):
    out_ref[...] = _layer_tail(aggp_ref, hs1_ref, dinv_ref, h0_ref, gb_ref,
                               wbd_ref, cb_ref, g_ref, b_ref, wrt_ref, rb_ref)


def _tc3(aggp1, hs1, dinv, h0, gb1, wbd1, cb1, g1, b1, wr1t, rb1):
    return pl.pallas_call(
        _tc3_body,
        compiler_params=_TC_PARAMS,
        out_shape=jax.ShapeDtypeStruct((_N, wbd1.shape[1]), jnp.float32),
    )(aggp1, hs1, dinv, h0, gb1, wbd1, cb1, g1, b1, wr1t, rb1)


def _blockdiag(wc):
    dout = wc.shape[0]
    g = dout // _H
    wg = wc.reshape(g, _H, _H)
    eye = jnp.eye(g, dtype=wc.dtype)
    bd = eye[:, None, :, None] * wg.transpose(0, 2, 1)[:, :, None, :]
    return bd.reshape(dout, dout)


def kernel(x, edge_index, W0, gb0, Wc0, cb0, g0, b0, Wr0, rb0,
           W1, gb1, Wc1, cb1, g1, b1, Wr1, rb1):
    e = edge_index.shape[1]
    per = _NW * _CH
    epad = _cdiv(e, per) * per
    pad = epad - e
    fill = jnp.full((pad,), _N, dtype=jnp.int32)
    srcp = jnp.concatenate([edge_index[0], fill])
    dstp = jnp.concatenate([edge_index[1], fill])

    zeros16 = jnp.zeros((_CH, 16), jnp.float32)
    ones16 = jnp.ones((_CH, 16), jnp.float32)
    zeros64 = jnp.zeros((_CH, 64), jnp.float32)
    zeros32 = jnp.zeros((_CH, 32), jnp.float32)

    row = lambda v: v.reshape(1, -1)
    wbd0 = _blockdiag(Wc0)
    wbd1 = _blockdiag(Wc1)

    degp = _sc_degree(dstp, zeros16, ones16, epad=epad)
    dinv, hs0 = _tc1(x, W0, degp)
    aggp0 = _sc_aggregate(hs0, srcp, dstp, zeros64, epad=epad, d=64)
    h0, hs1 = _tc2(aggp0, hs0, dinv, x, row(gb0), wbd0, row(cb0), row(g0),
                   row(b0), Wr0.T, row(rb0), W1)
    aggp1 = _sc_aggregate(hs1, srcp, dstp, zeros32, epad=epad, d=32)
    return _tc3(aggp1, hs1, dinv, h0, row(gb1), wbd1, row(cb1), row(g1),
                row(b1), Wr1.T, row(rb1))


# speedup vs baseline: 19.8462x; 1.2841x over previous
import functools

import jax
import jax.numpy as jnp
from jax import lax
from jax.experimental import pallas as pl
from jax.experimental.pallas import tpu as pltpu
from jax.experimental.pallas import tpu_sc as plsc

_N = 10000
_H = 4
_CH = 128
_CPW = 80
_NB = 5
_NC = 2
_NS = 16
_NW = _NC * _NS
_NPAD = 10112
_RPT = _NPAD // _NS
_HIGH = lax.Precision.HIGHEST
_TC_PARAMS = pltpu.CompilerParams(vmem_limit_bytes=100 * 1024 * 1024)


def _cdiv(a, b):
    return (a + b - 1) // b


def _sc_mesh():
    return plsc.VectorSubcoreMesh(core_axis_name="c", subcore_axis_name="s")


@functools.partial(jax.jit, static_argnames=("epad",))
def _sc_degree(dstp, zeros16, ones16, *, epad):
    @functools.partial(
        pl.kernel,
        out_type=jax.ShapeDtypeStruct((_NC, _NPAD, 16), jnp.float32),
        mesh=_sc_mesh(),
        compiler_params=pltpu.CompilerParams(use_tc_tiling_on_sc=False),
        scratch_types=[
            pltpu.VMEM_SHARED((_NPAD, 16), jnp.float32),
            pltpu.VMEM((_CH, 16), jnp.float32),
            pltpu.VMEM((_CH, 16), jnp.float32),
            pltpu.VMEM((_CPW, _CH), jnp.int32),
        ],
    )
    def degk(dst_hbm, z_hbm, o_hbm, out_hbm, acc_sh, zb, ob, didx):
        cid = lax.axis_index("c")
        sid = lax.axis_index("s")
        wid = cid * _NS + sid
        pltpu.sync_copy(dst_hbm.at[pl.ds(wid * _CPW, _CPW)], didx)
        pltpu.sync_copy(z_hbm, zb)
        pltpu.sync_copy(o_hbm, ob)
        for j, w in enumerate((128, 128, 128, 128, 120)):
            pltpu.sync_copy(
                zb.at[pl.ds(0, w)],
                acc_sh.at[pl.ds(sid * _RPT + j * 128, w)],
            )
        plsc.subcore_barrier()

        def body(j, carry):
            pltpu.sync_copy(ob, acc_sh.at[didx.at[j]], add=True)
            return carry

        lax.fori_loop(0, _CPW, body, 0)
        plsc.subcore_barrier()
        pltpu.sync_copy(
            acc_sh.at[pl.ds(sid * _RPT, _RPT)],
            out_hbm.at[cid, pl.ds(sid * _RPT, _RPT)],
        )

    return degk(dstp, zeros16, ones16)


@functools.partial(jax.jit, static_argnames=("epad", "d"))
def _sc_aggregate(hs, srcp, dstp, zrows, *, epad, d):
    @functools.partial(
        pl.kernel,
        out_type=jax.ShapeDtypeStruct((_NC, _NPAD, d), jnp.float32),
        mesh=_sc_mesh(),
        compiler_params=pltpu.CompilerParams(use_tc_tiling_on_sc=False),
        scratch_types=[
            pltpu.VMEM_SHARED((_NPAD, d), jnp.float32),
            pltpu.VMEM((_CH, d), jnp.float32),
            pltpu.VMEM((_NB, _CH, d), jnp.float32),
            pltpu.VMEM((_CPW, _CH), jnp.int32),
            pltpu.VMEM((_CPW, _CH), jnp.int32),
            pltpu.SemaphoreType.DMA((_NB,)),
        ],
    )
    def aggk(hs_hbm, src_hbm, dst_hbm, z_hbm, out_hbm,
             acc_sh, zb, rows, sidx, didx, sems):
        cid = lax.axis_index("c")
        sid = lax.axis_index("s")
        wid = cid * _NS + sid
        pltpu.sync_copy(src_hbm.at[pl.ds(wid * _CPW, _CPW)], sidx)
        pltpu.sync_copy(dst_hbm.at[pl.ds(wid * _CPW, _CPW)], didx)
        pltpu.sync_copy(z_hbm, zb)
        for j, w in enumerate((128, 128, 128, 128, 120)):
            pltpu.sync_copy(
                zb.at[pl.ds(0, w)],
                acc_sh.at[pl.ds(sid * _RPT + j * 128, w)],
            )
        plsc.subcore_barrier()

        for b in range(_NB):
            pltpu.async_copy(hs_hbm.at[sidx.at[b]], rows.at[b], sems.at[b])

        def group(g, carry):
            for b in range(_NB):
                j = g * _NB + b
                pltpu.make_async_copy(
                    hs_hbm.at[sidx.at[j]], rows.at[b], sems.at[b]).wait()
                pltpu.sync_copy(rows.at[b], acc_sh.at[didx.at[j]], add=True)
                pltpu.async_copy(
                    hs_hbm.at[sidx.at[j + _NB]], rows.at[b], sems.at[b])
            return carry

        lax.fori_loop(0, _CPW // _NB - 1, group, 0)
        tail = _CPW - _NB
        for b in range(_NB):
            pltpu.make_async_copy(
                hs_hbm.at[sidx.at[tail + b]], rows.at[b], sems.at[b]).wait()
            pltpu.sync_copy(rows.at[b], acc_sh.at[didx.at[tail + b]], add=True)

        plsc.subcore_barrier()
        pltpu.sync_copy(
            acc_sh.at[pl.ds(sid * _RPT, _RPT)],
            out_hbm.at[cid, pl.ds(sid * _RPT, _RPT)],
        )

    return aggk(hs, srcp, dstp, zrows)


def _tc1_body(x_ref, w0_ref, degp_ref, dinv_ref, hs0_ref):
    deg = degp_ref[0][0:_N, 0:1] + degp_ref[1][0:_N, 0:1] + 1.0
    dinv = lax.rsqrt(deg)
    dinv_ref[...] = dinv
    h1 = jnp.dot(x_ref[...], w0_ref[...], precision=_HIGH)
    hs0_ref[0:_N, :] = h1 * dinv


def _tc1(x, w0, degp):
    return pl.pallas_call(
        _tc1_body,
        compiler_params=_TC_PARAMS,
        out_shape=[
            jax.ShapeDtypeStruct((_N, 1), jnp.float32),
            jax.ShapeDtypeStruct((_N + 1, w0.shape[1]), jnp.float32),
        ],
    )(x, w0, degp)


def _layer_tail(aggp_ref, hs_ref, dinv_ref, res_in_ref, gb_ref, wbd_ref,
                cb_ref, g_ref, b_ref, wrt_ref, rb_ref):
    dinv = dinv_ref[...]
    agg = aggp_ref[0][0:_N, :] + aggp_ref[1][0:_N, :] + hs_ref[0:_N, :]
    h = dinv * agg + gb_ref[...]
    h = jnp.maximum(h, 0.0)
    hc = jnp.dot(h, wbd_ref[...], precision=_HIGH) + cb_ref[...]
    m = jnp.mean(hc, axis=0, keepdims=True)
    dlt = hc - m
    v = jnp.mean(dlt * dlt, axis=0, keepdims=True)
    hb = g_ref[...] * (dlt / jnp.sqrt(v + 1e-5)) + b_ref[...]
    res = jnp.dot(res_in_ref[...], wrt_ref[...], precision=_HIGH) + rb_ref[...]
    return hb + res


def _tc2_body(aggp_ref, hs0_ref, dinv_ref, x_ref, gb_ref, wbd_ref, cb_ref,
              g_ref, b_ref, wrt_ref, rb_ref, w1_ref, h0_ref, hs1_ref):
    h0 = _layer_tail(aggp_ref, hs0_ref, dinv_ref, x_ref, gb_ref, wbd_ref,
                     cb_ref, g_ref, b_ref, wrt_ref, rb_ref)
    h0_ref[...] = h0
    h1 = jnp.dot(h0, w1_ref[...], precision=_HIGH)
    hs1_ref[0:_N, :] = h1 * dinv_ref[...]


def _tc2(aggp0, hs0, dinv, x, gb0, wbd0, cb0, g0, b0, wr0t, rb0, w1):
    return pl.pallas_call(
        _tc2_body,
        compiler_params=_TC_PARAMS,
        out_shape=[
            jax.ShapeDtypeStruct((_N, wbd0.shape[1]), jnp.float32),
            jax.ShapeDtypeStruct((_N + 1, w1.shape[1]), jnp.float32),
        ],
    )(aggp0, hs0, dinv, x, gb0, wbd0, cb0, g0, b0, wr0t, rb0, w1)


def _tc3_body(aggp_ref, hs1_ref, dinv_ref, h0_ref, gb_ref, wbd_ref, cb_ref,
              g_ref, b_ref, wrt_ref, rb_ref, out_ref):
    out_ref[...] = _layer_tail(aggp_ref, hs1_ref, dinv_ref, h0_ref, gb_ref,
                               wbd_ref, cb_ref, g_ref, b_ref, wrt_ref, rb_ref)


def _tc3(aggp1, hs1, dinv, h0, gb1, wbd1, cb1, g1, b1, wr1t, rb1):
    return pl.pallas_call(
        _tc3_body,
        compiler_params=_TC_PARAMS,
        out_shape=jax.ShapeDtypeStruct((_N, wbd1.shape[1]), jnp.float32),
    )(aggp1, hs1, dinv, h0, gb1, wbd1, cb1, g1, b1, wr1t, rb1)


def _blockdiag(wc):
    dout = wc.shape[0]
    g = dout // _H
    wg = wc.reshape(g, _H, _H)
    eye = jnp.eye(g, dtype=wc.dtype)
    bd = eye[:, None, :, None] * wg.transpose(0, 2, 1)[:, :, None, :]
    return bd.reshape(dout, dout)


def kernel(x, edge_index, W0, gb0, Wc0, cb0, g0, b0, Wr0, rb0,
           W1, gb1, Wc1, cb1, g1, b1, Wr1, rb1):
    e = edge_index.shape[1]
    epad = _NW * _CPW * _CH
    pad = epad - e
    fill_dst = _N + (jnp.arange(pad, dtype=jnp.int32) % (_NPAD - _N))
    srcp = jnp.concatenate(
        [edge_index[0], jnp.zeros((pad,), jnp.int32)]).reshape(-1, _CH)
    dstp = jnp.concatenate([edge_index[1], fill_dst]).reshape(-1, _CH)

    zeros16 = jnp.zeros((_CH, 16), jnp.float32)
    ones16 = jnp.ones((_CH, 16), jnp.float32)
    zeros64 = jnp.zeros((_CH, 64), jnp.float32)
    zeros32 = jnp.zeros((_CH, 32), jnp.float32)

    row = lambda v: v.reshape(1, -1)
    wbd0 = _blockdiag(Wc0)
    wbd1 = _blockdiag(Wc1)

    degp = _sc_degree(dstp, zeros16, ones16, epad=epad)
    dinv, hs0 = _tc1(x, W0, degp)
    aggp0 = _sc_aggregate(hs0, srcp, dstp, zeros64, epad=epad, d=64)
    h0, hs1 = _tc2(aggp0, hs0, dinv, x, row(gb0), wbd0, row(cb0), row(g0),
                   row(b0), Wr0.T, row(rb0), W1)
    aggp1 = _sc_aggregate(hs1, srcp, dstp, zeros32, epad=epad, d=32)
    return _tc3(aggp1, hs1, dinv, h0, row(gb1), wbd1, row(cb1), row(g1),
                row(b1), Wr1.T, row(rb1))
